# baseline (speedup 1.0000x reference)
import functools

import jax
import jax.numpy as jnp
from jax import lax
from jax.experimental import pallas as pl
from jax.experimental.pallas import tpu as pltpu
from jax.experimental.pallas import tpu_sc as plsc

VOCAB = 1000000
D = 64
B = 16384
H = 200
HP = 208
NC, NS, L = 2, 16, 16
NW = NC * NS
ROWS_PER_W = B // NW


def _bow_body(x_hbm, table_hbm, out_hbm, idx_v, rows_v, out_v, sem0, sem1):
    wid = lax.axis_index("s") * NC + lax.axis_index("c")
    base = wid * ROWS_PER_W

    idx_v[pl.ds(192, 16)] = jnp.zeros((L,), jnp.int32)

    zero = jnp.zeros((L,), jnp.float32)

    def row_body(i, carry):
        row = base + i
        pltpu.sync_copy(x_hbm.at[row], idx_v.at[pl.ds(0, H)])
        cp0 = pltpu.async_copy(
            table_hbm.at[idx_v.at[pl.ds(0, 104)]], rows_v.at[pl.ds(0, 104)], sem0)
        cp1 = pltpu.async_copy(
            table_hbm.at[idx_v.at[pl.ds(104, 104)]], rows_v.at[pl.ds(104, 104)], sem1)

        def cnt_body(c, cnt):
            v = idx_v[pl.ds(c * L, L)]
            return cnt + jnp.where(v != 0, 1.0, 0.0)

        cntf = lax.fori_loop(0, HP // L, cnt_body, zero)
        total = lax.broadcast(jnp.sum(cntf), (L,))
        inv = jnp.ones((L,), jnp.float32) / total

        cp0.wait()
        cp1.wait()

        def acc_body(r, accs):
            a0, a1, a2, a3 = accs
            a0 = a0 + rows_v[r, pl.ds(0, L)]
            a1 = a1 + rows_v[r, pl.ds(L, L)]
            a2 = a2 + rows_v[r, pl.ds(2 * L, L)]
            a3 = a3 + rows_v[r, pl.ds(3 * L, L)]
            return (a0, a1, a2, a3)

        a0, a1, a2, a3 = lax.fori_loop(0, HP, acc_body, (zero, zero, zero, zero))

        out_v[pl.ds(0, L)] = a0 * inv
        out_v[pl.ds(L, L)] = a1 * inv
        out_v[pl.ds(2 * L, L)] = a2 * inv
        out_v[pl.ds(3 * L, L)] = a3 * inv
        pltpu.sync_copy(out_v, out_hbm.at[row])
        return carry

    lax.fori_loop(0, ROWS_PER_W, row_body, 0)


@functools.partial(jax.jit, static_argnames=())
def kernel(x, table):
    bow = pl.kernel(
        _bow_body,
        out_type=jax.ShapeDtypeStruct((B, D), jnp.float32),
        mesh=plsc.VectorSubcoreMesh(core_axis_name="c", subcore_axis_name="s"),
        compiler_params=pltpu.CompilerParams(
            needs_layout_passes=False, use_tc_tiling_on_sc=False),
        scratch_types=[
            pltpu.VMEM((HP,), jnp.int32),
            pltpu.VMEM((HP, D), jnp.float32),
            pltpu.VMEM((D,), jnp.float32),
            pltpu.SemaphoreType.DMA,
            pltpu.SemaphoreType.DMA,
        ],
    )
    return bow(x.astype(jnp.int32), table)


# speedup vs baseline: 3.7238x; 3.7238x over previous
import functools

import jax
import jax.numpy as jnp
from jax import lax
from jax.experimental import pallas as pl
from jax.experimental.pallas import tpu as pltpu
from jax.experimental.pallas import tpu_sc as plsc

VOCAB = 1000000
D = 64
B = 16384
H = 200
L = 16
NC, NS = 2, 16
NW = NC * NS
ROWS_PER_W = B // NW
G = 4
NG = ROWS_PER_W // G
S0, S1 = 104, 96


def _bow_body(x_hbm, table_hbm, out_hbm,
              idx0, idx1, rows0, rows1, ob0, ob1,
              isem0, isem1, gsem0, gsem1, osem0, osem1):
    wid = lax.axis_index("s") * NC + lax.axis_index("c")
    base = wid * ROWS_PER_W

    zero = jnp.zeros((L,), jnp.float32)
    lane = lax.iota(jnp.int32, L)

    def issue_idx(g, idxS, isemS):
        return pltpu.async_copy(x_hbm.at[pl.ds(base + g * G, G)], idxS, isemS)

    def wait_idx(g, idxS, isemS):
        pltpu.make_async_copy(x_hbm.at[pl.ds(base + g * G, G)], idxS, isemS).wait()

    def issue_gathers(idxS, rowsS, gsemS):
        for r in range(G):
            pltpu.async_copy(table_hbm.at[idxS.at[r, pl.ds(0, S0)]],
                             rowsS.at[r, pl.ds(0, S0)], gsemS)
            pltpu.async_copy(table_hbm.at[idxS.at[r, pl.ds(S0, S1)]],
                             rowsS.at[r, pl.ds(S0, S1)], gsemS)

    def wait_gathers(idxS, rowsS, gsemS):
        for r in range(G):
            pltpu.make_async_copy(table_hbm.at[idxS.at[r, pl.ds(0, S0)]],
                                  rowsS.at[r, pl.ds(0, S0)], gsemS).wait()
            pltpu.make_async_copy(table_hbm.at[idxS.at[r, pl.ds(S0, S1)]],
                                  rowsS.at[r, pl.ds(S0, S1)], gsemS).wait()

    def row_inv(idxS, r):
        cnt = zero
        for c in range(12):
            cnt = cnt + jnp.where(idxS[r, pl.ds(c * L, L)] != 0, 1.0, 0.0)
        tail = idxS[r, pl.ds(H - L, L)]
        cnt = cnt + jnp.where((tail != 0) & (lane >= 8), 1.0, 0.0)
        total = lax.broadcast(jnp.sum(cnt), (L,))
        return jnp.ones((L,), jnp.float32) / total

    def compute_group(invs, rowsS, obS):
        for r in range(G):
            inv = invs[r]

            def acc_body(t, accs, _r=r):
                a0, a1, a2, a3 = accs
                for u in range(4):
                    row = t * 4 + u
                    a0 = a0 + rowsS[_r, row, pl.ds(0, L)]
                    a1 = a1 + rowsS[_r, row, pl.ds(L, L)]
                    a2 = a2 + rowsS[_r, row, pl.ds(2 * L, L)]
                    a3 = a3 + rowsS[_r, row, pl.ds(3 * L, L)]
                return (a0, a1, a2, a3)

            a0, a1, a2, a3 = lax.fori_loop(0, H // 4, acc_body,
                                           (zero, zero, zero, zero))
            obS[r, pl.ds(0, L)] = a0 * inv
            obS[r, pl.ds(L, L)] = a1 * inv
            obS[r, pl.ds(2 * L, L)] = a2 * inv
            obS[r, pl.ds(3 * L, L)] = a3 * inv

    def issue_out(g, obS, osemS):
        return pltpu.async_copy(obS, out_hbm.at[pl.ds(base + g * G, G)], osemS)

    def drain_out(g, obS, osemS):
        pltpu.make_async_copy(obS, out_hbm.at[pl.ds(base + g * G, G)], osemS).wait()

    pltpu.sync_copy(x_hbm.at[pl.ds(base, G)], idx0)
    issue_gathers(idx0, rows0, gsem0)
    issue_idx(1, idx1, isem1)

    def body(k, carry):
        gA = 2 * k
        gB = 2 * k + 1

        wait_idx(gB, idx1, isem1)
        issue_gathers(idx1, rows1, gsem1)
        invsA = [row_inv(idx0, r) for r in range(G)]

        wait_gathers(idx0, rows0, gsem0)

        @pl.when(k < NG // 2 - 1)
        def _():
            issue_idx(gA + 2, idx0, isem0)

        @pl.when(k > 0)
        def _():
            drain_out(gA - 2, ob0, osem0)

        compute_group(invsA, rows0, ob0)
        issue_out(gA, ob0, osem0)

        invsB = [row_inv(idx1, r) for r in range(G)]

        @pl.when(k < NG // 2 - 1)
        def _():
            wait_idx(gA + 2, idx0, isem0)
            issue_gathers(idx0, rows0, gsem0)

        wait_gathers(idx1, rows1, gsem1)

        @pl.when(k < NG // 2 - 1)
        def _():
            issue_idx(gB + 2, idx1, isem1)

        @pl.when(k > 0)
        def _():
            drain_out(gB - 2, ob1, osem1)

        compute_group(invsB, rows1, ob1)
        issue_out(gB, ob1, osem1)
        return carry

    lax.fori_loop(0, NG // 2, body, 0)

    drain_out(NG - 2, ob0, osem0)
    drain_out(NG - 1, ob1, osem1)


@functools.partial(jax.jit, static_argnames=())
def kernel(x, table):
    bow = pl.kernel(
        _bow_body,
        out_type=jax.ShapeDtypeStruct((B, D), jnp.float32),
        mesh=plsc.VectorSubcoreMesh(core_axis_name="c", subcore_axis_name="s"),
        compiler_params=pltpu.CompilerParams(
            needs_layout_passes=False, use_tc_tiling_on_sc=False),
        scratch_types=[
            pltpu.VMEM((G, H), jnp.int32),
            pltpu.VMEM((G, H), jnp.int32),
            pltpu.VMEM((G, H, D), jnp.float32),
            pltpu.VMEM((G, H, D), jnp.float32),
            pltpu.VMEM((G, D), jnp.float32),
            pltpu.VMEM((G, D), jnp.float32),
            pltpu.SemaphoreType.DMA,
            pltpu.SemaphoreType.DMA,
            pltpu.SemaphoreType.DMA,
            pltpu.SemaphoreType.DMA,
            pltpu.SemaphoreType.DMA,
            pltpu.SemaphoreType.DMA,
        ],
    )
    return bow(x.astype(jnp.int32), table)


# speedup vs baseline: 3.9616x; 1.0639x over previous
import functools

import jax
import jax.numpy as jnp
from jax import lax
from jax.experimental import pallas as pl
from jax.experimental.pallas import tpu as pltpu
from jax.experimental.pallas import tpu_sc as plsc

VOCAB = 1000000
D = 64
B = 16384
H = 200
HH = H // 2
L = 16
NC, NS = 2, 16
NW = NC * NS
ROWS_PER_W = B // NW
G = 2
GI = 2 * G
NG = ROWS_PER_W // G
KB = 16
NB = NG // KB
NQ = KB // 4


def _bow_body(x_hbm, table_hbm, out_hbm,
              idxb0, idxb1, rows0, rows1, rows2, rows3, ob0, ob1,
              isem0, isem1, gsem0, gsem1, gsem2, gsem3, osem0, osem1):
    wid = lax.axis_index("s") * NC + lax.axis_index("c")
    base = wid * ROWS_PER_W

    rows = (rows0, rows1, rows2, rows3)
    gsems = (gsem0, gsem1, gsem2, gsem3)

    zero = jnp.zeros((L,), jnp.float32)
    lane = lax.iota(jnp.int32, L)

    def stage_idx(b, I, isemS):
        return pltpu.async_copy(
            x_hbm.at[pl.ds((base + b * KB * G) * 2, KB * GI)], I, isemS)

    def wait_stage(b, I, isemS):
        pltpu.make_async_copy(
            x_hbm.at[pl.ds((base + b * KB * G) * 2, KB * GI)], I, isemS).wait()

    def issue_g(I, p, R, gsemS):
        for h in range(GI):
            pltpu.async_copy(table_hbm.at[I.at[p * GI + h]], R.at[h], gsemS)

    def wait_g(I, p, R, gsemS):
        for h in range(GI):
            pltpu.make_async_copy(table_hbm.at[I.at[p * GI + h]],
                                  R.at[h], gsemS).wait()

    def half_cnt(I, h):
        cnt = zero
        for c in range(6):
            cnt = cnt + jnp.where(I[h, pl.ds(c * L, L)] != 0, 1.0, 0.0)
        tail = I[h, pl.ds(HH - L, L)]
        return cnt + jnp.where((tail != 0) & (lane >= 12), 1.0, 0.0)

    def compute_group(I, p, R, O):
        for rr in range(G):
            cnt = half_cnt(I, p * GI + 2 * rr) + half_cnt(I, p * GI + 2 * rr + 1)
            total = lax.broadcast(jnp.sum(cnt), (L,))
            inv = jnp.ones((L,), jnp.float32) / total

            def acc_body(t, accs, _rr=rr):
                a0, a1, a2, a3 = accs
                for uu in range(4):
                    tt = t * 4 + uu
                    for hh in range(2):
                        a0 = a0 + R[2 * _rr + hh, tt, pl.ds(0, L)]
                        a1 = a1 + R[2 * _rr + hh, tt, pl.ds(L, L)]
                        a2 = a2 + R[2 * _rr + hh, tt, pl.ds(2 * L, L)]
                        a3 = a3 + R[2 * _rr + hh, tt, pl.ds(3 * L, L)]
                return (a0, a1, a2, a3)

            a0, a1, a2, a3 = lax.fori_loop(0, HH // 4, acc_body,
                                           (zero, zero, zero, zero))
            orow = p * G + rr
            O[orow, pl.ds(0, L)] = a0 * inv
            O[orow, pl.ds(L, L)] = a1 * inv
            O[orow, pl.ds(2 * L, L)] = a2 * inv
            O[orow, pl.ds(3 * L, L)] = a3 * inv

    def issue_out(b, O, osemS):
        return pltpu.async_copy(
            O, out_hbm.at[pl.ds(base + b * KB * G, KB * G)], osemS)

    def drain_out(b, O, osemS):
        pltpu.make_async_copy(
            O, out_hbm.at[pl.ds(base + b * KB * G, KB * G)], osemS).wait()

    def process_block(b, I, Inext, isem_next, O, osemS):
        @pl.when(b + 1 < NB)
        def _():
            stage_idx(b + 1, Inext, isem_next)

        @pl.when(b >= 2)
        def _():
            drain_out(b - 2, O, osemS)

        def quad(m, carry):
            for u in range(4):
                p = 4 * m + u
                issue_g(I, p + 3, rows[(u + 3) % 4], gsems[(u + 3) % 4])
                wait_g(I, p, rows[u], gsems[u])
                compute_group(I, p, rows[u], O)
            return carry

        lax.fori_loop(0, NQ - 1, quad, 0)

        p0 = 4 * (NQ - 1)
        issue_g(I, p0 + 3, rows[3], gsems[3])
        wait_g(I, p0, rows[0], gsems[0])
        compute_group(I, p0, rows[0], O)

        @pl.when(b + 1 < NB)
        def _():
            wait_stage(b + 1, Inext, isem_next)
            issue_g(Inext, 0, rows[0], gsems[0])
        wait_g(I, p0 + 1, rows[1], gsems[1])
        compute_group(I, p0 + 1, rows[1], O)

        @pl.when(b + 1 < NB)
        def _():
            issue_g(Inext, 1, rows[1], gsems[1])
        wait_g(I, p0 + 2, rows[2], gsems[2])
        compute_group(I, p0 + 2, rows[2], O)

        @pl.when(b + 1 < NB)
        def _():
            issue_g(Inext, 2, rows[2], gsems[2])
        wait_g(I, p0 + 3, rows[3], gsems[3])
        compute_group(I, p0 + 3, rows[3], O)

        issue_out(b, O, osemS)

    pltpu.sync_copy(x_hbm.at[pl.ds(base * 2, KB * GI)], idxb0)
    issue_g(idxb0, 0, rows0, gsem0)
    issue_g(idxb0, 1, rows1, gsem1)
    issue_g(idxb0, 2, rows2, gsem2)

    def body(s, carry):
        process_block(2 * s, idxb0, idxb1, isem1, ob0, osem0)
        process_block(2 * s + 1, idxb1, idxb0, isem0, ob1, osem1)
        return carry

    lax.fori_loop(0, NB // 2, body, 0)

    drain_out(NB - 2, ob0, osem0)
    drain_out(NB - 1, ob1, osem1)


@functools.partial(jax.jit, static_argnames=())
def kernel(x, table):
    bow = pl.kernel(
        _bow_body,
        out_type=jax.ShapeDtypeStruct((B, D), jnp.float32),
        mesh=plsc.VectorSubcoreMesh(core_axis_name="c", subcore_axis_name="s"),
        compiler_params=pltpu.CompilerParams(
            needs_layout_passes=False, use_tc_tiling_on_sc=False),
        scratch_types=[
            pltpu.VMEM((KB * GI, HH), jnp.int32),
            pltpu.VMEM((KB * GI, HH), jnp.int32),
            pltpu.VMEM((GI, HH, D), jnp.float32),
            pltpu.VMEM((GI, HH, D), jnp.float32),
            pltpu.VMEM((GI, HH, D), jnp.float32),
            pltpu.VMEM((GI, HH, D), jnp.float32),
            pltpu.VMEM((KB * G, D), jnp.float32),
            pltpu.VMEM((KB * G, D), jnp.float32),
            pltpu.SemaphoreType.DMA,
            pltpu.SemaphoreType.DMA,
            pltpu.SemaphoreType.DMA,
            pltpu.SemaphoreType.DMA,
            pltpu.SemaphoreType.DMA,
            pltpu.SemaphoreType.DMA,
            pltpu.SemaphoreType.DMA,
            pltpu.SemaphoreType.DMA,
        ],
    )
    x2 = x.astype(jnp.int32).reshape(2 * B, HH)
    return bow(x2, table)
